# baseline (speedup 1.0000x reference)
import functools

import jax
import jax.numpy as jnp
from jax import lax
from jax.experimental import pallas as pl
from jax.experimental.pallas import tpu as pltpu
from jax.experimental.pallas import tpu_sc as plsc

E = 8
D = 1024
DFF = 1024
T = 2048
K = 2

BLK = 256
NSLOT = 3
P = T * K + E * BLK
NBLOCKS = P // BLK

NC = 2
NS = 16
NW = NC * NS
TPW = T // NW
PPW = (T * K) // NW
WREP_W = 128

def _worker_id():
    return lax.axis_index("s") * NC + lax.axis_index("c")


@functools.lru_cache(maxsize=None)
def _sc_mesh():
    return plsc.VectorSubcoreMesh(core_axis_name="c", subcore_axis_name="s",
                                  num_cores=NC, num_subcores=NS)


@functools.lru_cache(maxsize=None)
def _sc_dispatch():
    @functools.partial(
        pl.kernel,
        out_type=(
            jax.ShapeDtypeStruct((P, D), jnp.float32),
            jax.ShapeDtypeStruct((P, WREP_W), jnp.float32),
        ),
        mesh=_sc_mesh(),
        scratch_types=(
            pltpu.VMEM((TPW, D), jnp.float32),
            pltpu.VMEM((PPW, WREP_W), jnp.float32),
            pltpu.VMEM((TPW,), jnp.int32),
            pltpu.VMEM((TPW,), jnp.int32),
            pltpu.VMEM((PPW,), jnp.int32),
            pltpu.SemaphoreType.DMA,
        ),
    )
    def body(hidden_hbm, wrep_hbm, pos_e_hbm, pos_o_hbm, pos_hbm,
             xg_hbm, wg_hbm, tbuf, wbuf, idx_e, idx_o, idx_p, sem):
        wid = _worker_id()
        tb = wid * TPW
        pb = wid * PPW
        pltpu.sync_copy(hidden_hbm.at[pl.ds(tb, TPW)], tbuf)
        pltpu.sync_copy(pos_e_hbm.at[pl.ds(tb, TPW)], idx_e)
        pltpu.sync_copy(pos_o_hbm.at[pl.ds(tb, TPW)], idx_o)
        pltpu.async_copy(tbuf, xg_hbm.at[idx_e], sem).wait()
        pltpu.async_copy(tbuf, xg_hbm.at[idx_o], sem).wait()
        pltpu.sync_copy(wrep_hbm.at[pl.ds(pb, PPW)], wbuf)
        pltpu.sync_copy(pos_hbm.at[pl.ds(pb, PPW)], idx_p)
        pltpu.async_copy(wbuf, wg_hbm.at[idx_p], sem).wait()

    return body


@functools.lru_cache(maxsize=None)
def _sc_combine():
    @functools.partial(
        pl.kernel,
        out_type=(
            jax.ShapeDtypeStruct((T, D), jnp.float32),
            jax.ShapeDtypeStruct((T, D), jnp.float32),
        ),
        mesh=_sc_mesh(),
        scratch_types=(
            pltpu.VMEM((TPW, D), jnp.float32),
            pltpu.VMEM((TPW,), jnp.int32),
            pltpu.SemaphoreType.DMA,
        ),
    )
    def body(yg_hbm, pos_e_hbm, pos_o_hbm, y0_hbm, y1_hbm, buf, idx, sem):
        wid = _worker_id()
        tb = wid * TPW
        pltpu.sync_copy(pos_e_hbm.at[pl.ds(tb, TPW)], idx)
        pltpu.async_copy(yg_hbm.at[idx], buf, sem).wait()
        pltpu.sync_copy(buf, y0_hbm.at[pl.ds(tb, TPW)])
        pltpu.sync_copy(pos_o_hbm.at[pl.ds(tb, TPW)], idx)
        pltpu.async_copy(yg_hbm.at[idx], buf, sem).wait()
        pltpu.sync_copy(buf, y1_hbm.at[pl.ds(tb, TPW)])

    return body


def _mlp_body(be_ref, nv_ref, jb_ref, seq_ref, nreg_ref,
              x_ref, gup_any, dwn_any, wp_ref, y_ref,
              gup_sl, dwn_sl, gsem, dsem):
    i = pl.program_id(0)
    valid = i < nv_ref[0]
    j = jb_ref[i]
    first = jnp.logical_or(i == 0, be_ref[i] != be_ref[jnp.maximum(i - 1, 0)])

    def _start(jj):
        sl = lax.rem(jj, NSLOT)
        e = seq_ref[jj]
        pltpu.make_async_copy(gup_any.at[e], gup_sl.at[sl], gsem.at[sl]).start()
        pltpu.make_async_copy(dwn_any.at[e], dwn_sl.at[sl], dsem.at[sl]).start()

    def _wait(jj):
        sl = lax.rem(jj, NSLOT)
        e = seq_ref[jj]
        pltpu.make_async_copy(gup_any.at[e], gup_sl.at[sl], gsem.at[sl]).wait()
        pltpu.make_async_copy(dwn_any.at[e], dwn_sl.at[sl], dsem.at[sl]).wait()

    @pl.when(i == 0)
    def _():
        _start(0)
        for jj in range(1, NSLOT):
            @pl.when(nreg_ref[0] > jj)
            def _(jj=jj):
                _start(jj)

    @pl.when(jnp.logical_and(valid, first))
    def _():
        _wait(j)

        @pl.when(jnp.logical_and(j >= 1, j + NSLOT - 1 < nreg_ref[0]))
        def _():
            _start(j + NSLOT - 1)

    @pl.when(valid)
    def _():
        sl = lax.rem(j, NSLOT)
        x = x_ref[...]
        h = lax.dot_general(x, gup_sl[sl], (((1,), (1,)), ((), ())),
                            preferred_element_type=jnp.float32,
                            precision=lax.Precision.DEFAULT)
        gate = h[:, :DFF]
        up = h[:, DFF:]
        act = gate * lax.logistic(gate) * up
        y = lax.dot_general(act, dwn_sl[sl], (((1,), (1,)), ((), ())),
                            preferred_element_type=jnp.float32,
                            precision=lax.Precision.DEFAULT)
        y_ref[...] = y * wp_ref[:, 0:1]


def _grouped_mlp(block_expert, nvalid, jb, seq, nreg, xg, gup, dwn, wg):
    grid_spec = pltpu.PrefetchScalarGridSpec(
        num_scalar_prefetch=5,
        grid=(NBLOCKS,),
        in_specs=[
            pl.BlockSpec((BLK, D), lambda i, *_: (i, 0)),
            pl.BlockSpec(memory_space=pl.ANY),
            pl.BlockSpec(memory_space=pl.ANY),
            pl.BlockSpec((BLK, WREP_W), lambda i, *_: (i, 0)),
        ],
        out_specs=pl.BlockSpec((BLK, D), lambda i, *_: (i, 0)),
        scratch_shapes=[
            pltpu.VMEM((NSLOT, 2 * DFF, D), jnp.float32),
            pltpu.VMEM((NSLOT, D, DFF), jnp.float32),
            pltpu.SemaphoreType.DMA((NSLOT,)),
            pltpu.SemaphoreType.DMA((NSLOT,)),
        ],
    )
    return pl.pallas_call(
        _mlp_body,
        grid_spec=grid_spec,
        out_shape=jax.ShapeDtypeStruct((P, D), jnp.float32),
    )(block_expert, nvalid, jb, seq, nreg, xg, gup, dwn, wg)


def _add_body(a_ref, b_ref, o_ref):
    o_ref[...] = a_ref[...] + b_ref[...]


def _combine_add(y0, y1):
    return pl.pallas_call(
        _add_body,
        grid=(T // 256,),
        in_specs=[
            pl.BlockSpec((256, D), lambda i: (i, 0)),
            pl.BlockSpec((256, D), lambda i: (i, 0)),
        ],
        out_specs=pl.BlockSpec((256, D), lambda i: (i, 0)),
        out_shape=jax.ShapeDtypeStruct((T, D), jnp.float32),
    )(y0, y1)


def kernel(hidden_states, top_k_index, top_k_weights, gate_up_proj, down_proj):
    orig_shape = hidden_states.shape
    x = hidden_states.reshape(-1, D)
    idx = top_k_index.reshape(-1, K).astype(jnp.int32)
    w = top_k_weights.reshape(-1, K).astype(jnp.float32)

    eflat = idx.reshape(-1)
    ohi = (eflat[:, None] == jnp.arange(E, dtype=jnp.int32)[None, :]).astype(jnp.int32)
    rank = jnp.sum((jnp.cumsum(ohi, axis=0) - ohi) * ohi, axis=1)
    counts = jnp.sum(ohi, axis=0)
    padded = ((counts + BLK - 1) // BLK) * BLK
    ends = jnp.cumsum(padded)
    starts = ends - padded
    pos = jnp.sum(ohi * starts[None, :], axis=1) + rank
    pos2 = pos.reshape(T, K)
    pos_e = pos2[:, 0]
    pos_o = pos2[:, 1]
    blk_start = jnp.arange(NBLOCKS, dtype=jnp.int32) * BLK
    valid_blk = blk_start < ends[-1]
    be_raw = jnp.minimum(
        jnp.sum((blk_start[:, None] >= ends[None, :]).astype(jnp.int32), axis=1),
        E - 1).astype(jnp.int32)
    be_last = jnp.max(jnp.where(valid_blk, be_raw, 0)).astype(jnp.int32)
    block_expert = jnp.where(valid_blk, be_raw, be_last)
    first_flag = jnp.concatenate([
        jnp.ones((1,), jnp.int32),
        (block_expert[1:] != block_expert[:-1]).astype(jnp.int32)])
    jb = jnp.cumsum(first_flag) - 1
    nreg = (jb[-1] + 1).reshape(1)
    seq = jnp.zeros((E,), jnp.int32).at[jb].max(block_expert)
    nvalid = (ends[-1] // BLK).astype(jnp.int32).reshape(1)
    wrep = jnp.broadcast_to(w.reshape(-1, 1), (T * K, WREP_W))

    xg, wg = _sc_dispatch()(x, wrep, pos_e, pos_o, pos)
    yg = _grouped_mlp(block_expert, nvalid, jb, seq, nreg, xg,
                      gate_up_proj, down_proj, wg)
    y0, y1 = _sc_combine()(yg, pos_e, pos_o)
    out = _combine_add(y0, y1)
    return out.reshape(orig_shape)


# speedup vs baseline: 1.0159x; 1.0159x over previous
import functools

import jax
import jax.numpy as jnp
from jax import lax
from jax.experimental import pallas as pl
from jax.experimental.pallas import tpu as pltpu
from jax.experimental.pallas import tpu_sc as plsc

E = 8
D = 1024
DFF = 1024
T = 2048
K = 2

BLK = 256
NSLOT = 2
P = T * K + E * BLK
NBLOCKS = P // BLK

NC = 2
NS = 16
NW = NC * NS
TPW = T // NW
PPW = (T * K) // NW
WREP_W = 128


def _worker_id():
    return lax.axis_index("s") * NC + lax.axis_index("c")


@functools.lru_cache(maxsize=None)
def _sc_mesh():
    return plsc.VectorSubcoreMesh(core_axis_name="c", subcore_axis_name="s",
                                  num_cores=NC, num_subcores=NS)


@functools.lru_cache(maxsize=None)
def _sc_dispatch():
    @functools.partial(
        pl.kernel,
        out_type=(
            jax.ShapeDtypeStruct((P, D), jnp.float32),
            jax.ShapeDtypeStruct((P, WREP_W), jnp.float32),
        ),
        mesh=_sc_mesh(),
        scratch_types=(
            pltpu.VMEM((TPW, D), jnp.float32),
            pltpu.VMEM((PPW, WREP_W), jnp.float32),
            pltpu.VMEM((TPW,), jnp.int32),
            pltpu.VMEM((TPW,), jnp.int32),
            pltpu.VMEM((PPW,), jnp.int32),
            pltpu.SemaphoreType.DMA,
        ),
    )
    def body(hidden_hbm, wrep_hbm, pos_e_hbm, pos_o_hbm, pos_hbm,
             xg_hbm, wg_hbm, tbuf, wbuf, idx_e, idx_o, idx_p, sem):
        wid = _worker_id()
        tb = wid * TPW
        pb = wid * PPW
        pltpu.sync_copy(hidden_hbm.at[pl.ds(tb, TPW)], tbuf)
        pltpu.sync_copy(pos_e_hbm.at[pl.ds(tb, TPW)], idx_e)
        pltpu.sync_copy(pos_o_hbm.at[pl.ds(tb, TPW)], idx_o)
        pltpu.async_copy(tbuf, xg_hbm.at[idx_e], sem).wait()
        pltpu.async_copy(tbuf, xg_hbm.at[idx_o], sem).wait()
        pltpu.sync_copy(wrep_hbm.at[pl.ds(pb, PPW)], wbuf)
        pltpu.sync_copy(pos_hbm.at[pl.ds(pb, PPW)], idx_p)
        pltpu.async_copy(wbuf, wg_hbm.at[idx_p], sem).wait()

    return body


@functools.lru_cache(maxsize=None)
def _sc_combine():
    @functools.partial(
        pl.kernel,
        out_type=(
            jax.ShapeDtypeStruct((T, D), jnp.float32),
            jax.ShapeDtypeStruct((T, D), jnp.float32),
        ),
        mesh=_sc_mesh(),
        scratch_types=(
            pltpu.VMEM((TPW, D), jnp.float32),
            pltpu.VMEM((TPW,), jnp.int32),
            pltpu.SemaphoreType.DMA,
        ),
    )
    def body(yg_hbm, pos_e_hbm, pos_o_hbm, y0_hbm, y1_hbm, buf, idx, sem):
        wid = _worker_id()
        tb = wid * TPW
        pltpu.sync_copy(pos_e_hbm.at[pl.ds(tb, TPW)], idx)
        pltpu.async_copy(yg_hbm.at[idx], buf, sem).wait()
        pltpu.sync_copy(buf, y0_hbm.at[pl.ds(tb, TPW)])
        pltpu.sync_copy(pos_o_hbm.at[pl.ds(tb, TPW)], idx)
        pltpu.async_copy(yg_hbm.at[idx], buf, sem).wait()
        pltpu.sync_copy(buf, y1_hbm.at[pl.ds(tb, TPW)])

    return body


def _mlp_body(be_ref, nv_ref, jb_ref, seq_ref, nreg_ref,
              x_ref, gup_any, dwn_any, wp_ref, y_ref,
              gup_sl, dwn_sl, gsem, gsem2, dsem):
    i = pl.program_id(0)
    valid = i < nv_ref[0]
    j = jb_ref[i]
    first = jnp.logical_or(i == 0, be_ref[i] != be_ref[jnp.maximum(i - 1, 0)])

    def _copies(jj):
        sl = lax.rem(jj, NSLOT)
        e = seq_ref[jj]
        return (
            pltpu.make_async_copy(gup_any.at[e, pl.ds(0, DFF)],
                                  gup_sl.at[sl, pl.ds(0, DFF)], gsem.at[sl]),
            pltpu.make_async_copy(gup_any.at[e, pl.ds(DFF, DFF)],
                                  gup_sl.at[sl, pl.ds(DFF, DFF)], gsem2.at[sl]),
            pltpu.make_async_copy(dwn_any.at[e], dwn_sl.at[sl], dsem.at[sl]),
        )

    def _start(jj):
        for c in _copies(jj):
            c.start()

    def _wait(jj):
        for c in _copies(jj):
            c.wait()

    @pl.when(i == 0)
    def _():
        _start(0)
        for jj in range(1, NSLOT):
            @pl.when(nreg_ref[0] > jj)
            def _(jj=jj):
                _start(jj)

    @pl.when(jnp.logical_and(valid, first))
    def _():
        _wait(j)

        @pl.when(jnp.logical_and(j >= 1, j + NSLOT - 1 < nreg_ref[0]))
        def _():
            _start(j + NSLOT - 1)

    @pl.when(valid)
    def _():
        sl = lax.rem(j, NSLOT)
        x = x_ref[...]
        h = lax.dot_general(x, gup_sl[sl], (((1,), (1,)), ((), ())),
                            preferred_element_type=jnp.float32,
                            precision=lax.Precision.DEFAULT)
        gate = h[:, :DFF]
        up = h[:, DFF:]
        act = gate * lax.logistic(gate) * up
        y = lax.dot_general(act, dwn_sl[sl], (((1,), (1,)), ((), ())),
                            preferred_element_type=jnp.float32,
                            precision=lax.Precision.DEFAULT)
        y_ref[...] = y * wp_ref[:, 0:1]


def _grouped_mlp(block_expert, nvalid, jb, seq, nreg, xg, gup, dwn, wg):
    grid_spec = pltpu.PrefetchScalarGridSpec(
        num_scalar_prefetch=5,
        grid=(NBLOCKS,),
        in_specs=[
            pl.BlockSpec((BLK, D), lambda i, *_: (i, 0)),
            pl.BlockSpec(memory_space=pl.ANY),
            pl.BlockSpec(memory_space=pl.ANY),
            pl.BlockSpec((BLK, WREP_W), lambda i, *_: (i, 0)),
        ],
        out_specs=pl.BlockSpec((BLK, D), lambda i, *_: (i, 0)),
        scratch_shapes=[
            pltpu.VMEM((NSLOT, 2 * DFF, D), jnp.float32),
            pltpu.VMEM((NSLOT, D, DFF), jnp.float32),
            pltpu.SemaphoreType.DMA((NSLOT,)),
            pltpu.SemaphoreType.DMA((NSLOT,)),
            pltpu.SemaphoreType.DMA((NSLOT,)),
        ],
    )
    return pl.pallas_call(
        _mlp_body,
        grid_spec=grid_spec,
        out_shape=jax.ShapeDtypeStruct((P, D), jnp.float32),
    )(block_expert, nvalid, jb, seq, nreg, xg, gup, dwn, wg)


def _add_body(a_ref, b_ref, o_ref):
    o_ref[...] = a_ref[...] + b_ref[...]


def _combine_add(y0, y1):
    return pl.pallas_call(
        _add_body,
        grid=(T // 256,),
        in_specs=[
            pl.BlockSpec((256, D), lambda i: (i, 0)),
            pl.BlockSpec((256, D), lambda i: (i, 0)),
        ],
        out_specs=pl.BlockSpec((256, D), lambda i: (i, 0)),
        out_shape=jax.ShapeDtypeStruct((T, D), jnp.float32),
    )(y0, y1)


def kernel(hidden_states, top_k_index, top_k_weights, gate_up_proj, down_proj):
    orig_shape = hidden_states.shape
    x = hidden_states.reshape(-1, D)
    idx = top_k_index.reshape(-1, K).astype(jnp.int32)
    w = top_k_weights.reshape(-1, K).astype(jnp.float32)

    eflat = idx.reshape(-1)
    ohi = (eflat[:, None] == jnp.arange(E, dtype=jnp.int32)[None, :]).astype(jnp.int32)
    rank = jnp.sum((jnp.cumsum(ohi, axis=0) - ohi) * ohi, axis=1)
    counts = jnp.sum(ohi, axis=0)
    padded = ((counts + BLK - 1) // BLK) * BLK
    ends = jnp.cumsum(padded)
    starts = ends - padded
    pos = jnp.sum(ohi * starts[None, :], axis=1) + rank
    pos2 = pos.reshape(T, K)
    pos_e = pos2[:, 0]
    pos_o = pos2[:, 1]
    blk_start = jnp.arange(NBLOCKS, dtype=jnp.int32) * BLK
    valid_blk = blk_start < ends[-1]
    be_raw = jnp.minimum(
        jnp.sum((blk_start[:, None] >= ends[None, :]).astype(jnp.int32), axis=1),
        E - 1).astype(jnp.int32)
    be_last = jnp.max(jnp.where(valid_blk, be_raw, 0)).astype(jnp.int32)
    block_expert = jnp.where(valid_blk, be_raw, be_last)
    first_flag = jnp.concatenate([
        jnp.ones((1,), jnp.int32),
        (block_expert[1:] != block_expert[:-1]).astype(jnp.int32)])
    jb = jnp.cumsum(first_flag) - 1
    nreg = (jb[-1] + 1).reshape(1)
    seq = jnp.zeros((E,), jnp.int32).at[jb].max(block_expert)
    nvalid = (ends[-1] // BLK).astype(jnp.int32).reshape(1)
    wrep = jnp.broadcast_to(w.reshape(-1, 1), (T * K, WREP_W))

    xg, wg = _sc_dispatch()(x, wrep, pos_e, pos_o, pos)
    yg = _grouped_mlp(block_expert, nvalid, jb, seq, nreg, xg,
                      gate_up_proj, down_proj, wg)
    y0, y1 = _sc_combine()(yg, pos_e, pos_o)
    out = _combine_add(y0, y1)
    return out.reshape(orig_shape)


# speedup vs baseline: 1.1061x; 1.0888x over previous
import functools

import jax
import jax.numpy as jnp
from jax import lax
from jax.experimental import pallas as pl
from jax.experimental.pallas import tpu as pltpu
from jax.experimental.pallas import tpu_sc as plsc

E = 8
D = 1024
DFF = 1024
T = 2048
K = 2

BLK = 256
NSLOT = 2
P = T * K + E * BLK
NBLOCKS = P // BLK

NC = 2
NS = 16
NW = NC * NS
TPW = T // NW
PPW = (T * K) // NW
WREP_W = 128
D2 = D // 2


def _worker_id():
    return lax.axis_index("s") * NC + lax.axis_index("c")


@functools.lru_cache(maxsize=None)
def _sc_mesh():
    return plsc.VectorSubcoreMesh(core_axis_name="c", subcore_axis_name="s",
                                  num_cores=NC, num_subcores=NS)


@functools.lru_cache(maxsize=None)
def _sc_dispatch():
    @functools.partial(
        pl.kernel,
        out_type=(
            jax.ShapeDtypeStruct((P, D2), jnp.float32),
            jax.ShapeDtypeStruct((P, WREP_W), jnp.float32),
        ),
        mesh=_sc_mesh(),
        scratch_types=(
            pltpu.VMEM((TPW, D2), jnp.float32),
            pltpu.VMEM((PPW, WREP_W), jnp.float32),
            pltpu.VMEM((TPW,), jnp.int32),
            pltpu.VMEM((TPW,), jnp.int32),
            pltpu.VMEM((PPW,), jnp.int32),
            pltpu.SemaphoreType.DMA,
        ),
    )
    def body(hidden_hbm, wrep_hbm, pos_e_hbm, pos_o_hbm, pos_hbm,
             xg_hbm, wg_hbm, tbuf, wbuf, idx_e, idx_o, idx_p, sem):
        wid = _worker_id()
        tb = wid * TPW
        pb = wid * PPW
        pltpu.sync_copy(hidden_hbm.at[pl.ds(tb, TPW)], tbuf)
        pltpu.sync_copy(pos_e_hbm.at[pl.ds(tb, TPW)], idx_e)
        pltpu.sync_copy(pos_o_hbm.at[pl.ds(tb, TPW)], idx_o)
        pltpu.async_copy(tbuf, xg_hbm.at[idx_e], sem).wait()
        pltpu.async_copy(tbuf, xg_hbm.at[idx_o], sem).wait()
        pltpu.sync_copy(wrep_hbm.at[pl.ds(pb, PPW)], wbuf)
        pltpu.sync_copy(pos_hbm.at[pl.ds(pb, PPW)], idx_p)
        pltpu.async_copy(wbuf, wg_hbm.at[idx_p], sem).wait()

    return body


@functools.lru_cache(maxsize=None)
def _sc_combine():
    @functools.partial(
        pl.kernel,
        out_type=(
            jax.ShapeDtypeStruct((T, D2), jnp.float32),
            jax.ShapeDtypeStruct((T, D2), jnp.float32),
        ),
        mesh=_sc_mesh(),
        scratch_types=(
            pltpu.VMEM((TPW, D2), jnp.float32),
            pltpu.VMEM((TPW,), jnp.int32),
            pltpu.SemaphoreType.DMA,
        ),
    )
    def body(yg_hbm, pos_e_hbm, pos_o_hbm, y0_hbm, y1_hbm, buf, idx, sem):
        wid = _worker_id()
        tb = wid * TPW
        pltpu.sync_copy(pos_e_hbm.at[pl.ds(tb, TPW)], idx)
        pltpu.async_copy(yg_hbm.at[idx], buf, sem).wait()
        pltpu.sync_copy(buf, y0_hbm.at[pl.ds(tb, TPW)])
        pltpu.sync_copy(pos_o_hbm.at[pl.ds(tb, TPW)], idx)
        pltpu.async_copy(yg_hbm.at[idx], buf, sem).wait()
        pltpu.sync_copy(buf, y1_hbm.at[pl.ds(tb, TPW)])

    return body


def _mlp_body(be_ref, nv_ref, jb_ref, seq_ref, nreg_ref,
              x_ref, gup_any, dwn_any, wp_ref, y_ref,
              gup_sl, dwn_sl, gsem, gsem2, dsem):
    i = pl.program_id(0)
    valid = i < nv_ref[0]
    j = jb_ref[i]
    first = jnp.logical_or(i == 0, be_ref[i] != be_ref[jnp.maximum(i - 1, 0)])

    def _copies(jj):
        sl = lax.rem(jj, NSLOT)
        e = seq_ref[jj]
        return (
            pltpu.make_async_copy(gup_any.at[e, pl.ds(0, DFF)],
                                  gup_sl.at[sl, pl.ds(0, DFF)], gsem.at[sl]),
            pltpu.make_async_copy(gup_any.at[e, pl.ds(DFF, DFF)],
                                  gup_sl.at[sl, pl.ds(DFF, DFF)], gsem2.at[sl]),
            pltpu.make_async_copy(dwn_any.at[e], dwn_sl.at[sl], dsem.at[sl]),
        )

    def _start(jj):
        for c in _copies(jj):
            c.start()

    def _wait(jj):
        for c in _copies(jj):
            c.wait()

    @pl.when(i == 0)
    def _():
        _start(0)
        for jj in range(1, NSLOT):
            @pl.when(nreg_ref[0] > jj)
            def _(jj=jj):
                _start(jj)

    @pl.when(jnp.logical_and(valid, first))
    def _():
        _wait(j)

        @pl.when(jnp.logical_and(j >= 1, j + NSLOT - 1 < nreg_ref[0]))
        def _():
            _start(j + NSLOT - 1)

    @pl.when(valid)
    def _():
        sl = lax.rem(j, NSLOT)
        xi = lax.bitcast_convert_type(x_ref[...], jnp.int32)
        xa = lax.bitcast_convert_type(lax.shift_left(xi, 16), jnp.float32)
        xb = lax.bitcast_convert_type(
            lax.bitwise_and(xi, jnp.int32(-65536)), jnp.float32)
        x = jnp.concatenate([xa, xb], axis=1)
        h = lax.dot_general(x, gup_sl[sl], (((1,), (1,)), ((), ())),
                            preferred_element_type=jnp.float32,
                            precision=lax.Precision.DEFAULT)
        gate = h[:, :DFF]
        up = h[:, DFF:]
        act = gate * lax.logistic(gate) * up
        y = lax.dot_general(act, dwn_sl[sl], (((1,), (1,)), ((), ())),
                            preferred_element_type=jnp.float32,
                            precision=lax.Precision.DEFAULT)
        yw = y * wp_ref[:, 0:1]
        ra = lax.bitcast_convert_type(
            yw[:, :D2].astype(jnp.bfloat16).astype(jnp.float32), jnp.int32)
        rb = lax.bitcast_convert_type(
            yw[:, D2:].astype(jnp.bfloat16).astype(jnp.float32), jnp.int32)
        y_ref[...] = lax.bitcast_convert_type(
            lax.shift_right_logical(ra, 16)
            | lax.bitwise_and(rb, jnp.int32(-65536)), jnp.float32)


def _grouped_mlp(block_expert, nvalid, jb, seq, nreg, xg, gup, dwn, wg):
    grid_spec = pltpu.PrefetchScalarGridSpec(
        num_scalar_prefetch=5,
        grid=(NBLOCKS,),
        in_specs=[
            pl.BlockSpec((BLK, D2),
                         lambda i, be, nv, *_: (jnp.minimum(i, nv[0] - 1), 0)),
            pl.BlockSpec(memory_space=pl.ANY),
            pl.BlockSpec(memory_space=pl.ANY),
            pl.BlockSpec((BLK, WREP_W),
                         lambda i, be, nv, *_: (jnp.minimum(i, nv[0] - 1), 0)),
        ],
        out_specs=pl.BlockSpec(
            (BLK, D2), lambda i, be, nv, *_: (jnp.minimum(i, nv[0] - 1), 0)),
        scratch_shapes=[
            pltpu.VMEM((NSLOT, 2 * DFF, D), jnp.float32),
            pltpu.VMEM((NSLOT, D, DFF), jnp.float32),
            pltpu.SemaphoreType.DMA((NSLOT,)),
            pltpu.SemaphoreType.DMA((NSLOT,)),
            pltpu.SemaphoreType.DMA((NSLOT,)),
        ],
    )
    return pl.pallas_call(
        _mlp_body,
        grid_spec=grid_spec,
        out_shape=jax.ShapeDtypeStruct((P, D2), jnp.float32),
    )(block_expert, nvalid, jb, seq, nreg, xg, gup, dwn, wg)


def _add_body(a_ref, b_ref, o_ref):
    q0 = lax.bitcast_convert_type(a_ref[...], jnp.int32)
    q1 = lax.bitcast_convert_type(b_ref[...], jnp.int32)

    def lo(q):
        return lax.bitcast_convert_type(lax.shift_left(q, 16), jnp.float32)

    def hi(q):
        return lax.bitcast_convert_type(
            lax.bitwise_and(q, jnp.int32(-65536)), jnp.float32)

    o_ref[:, :D2] = lo(q0) + lo(q1)
    o_ref[:, D2:] = hi(q0) + hi(q1)


def _combine_add(y0, y1):
    return pl.pallas_call(
        _add_body,
        grid=(T // 256,),
        in_specs=[
            pl.BlockSpec((256, D2), lambda i: (i, 0)),
            pl.BlockSpec((256, D2), lambda i: (i, 0)),
        ],
        out_specs=pl.BlockSpec((256, D), lambda i: (i, 0)),
        out_shape=jax.ShapeDtypeStruct((T, D), jnp.float32),
    )(y0, y1)


def kernel(hidden_states, top_k_index, top_k_weights, gate_up_proj, down_proj):
    orig_shape = hidden_states.shape
    x = hidden_states.reshape(-1, D)
    idx = top_k_index.reshape(-1, K).astype(jnp.int32)
    w = top_k_weights.reshape(-1, K).astype(jnp.float32)

    eflat = idx.reshape(-1)
    ohi = (eflat[:, None] == jnp.arange(E, dtype=jnp.int32)[None, :]).astype(jnp.int32)
    rank = jnp.sum((jnp.cumsum(ohi, axis=0) - ohi) * ohi, axis=1)
    counts = jnp.sum(ohi, axis=0)
    padded = ((counts + BLK - 1) // BLK) * BLK
    ends = jnp.cumsum(padded)
    starts = ends - padded
    pos = jnp.sum(ohi * starts[None, :], axis=1) + rank
    pos2 = pos.reshape(T, K)
    pos_e = pos2[:, 0]
    pos_o = pos2[:, 1]
    blk_start = jnp.arange(NBLOCKS, dtype=jnp.int32) * BLK
    valid_blk = blk_start < ends[-1]
    be_raw = jnp.minimum(
        jnp.sum((blk_start[:, None] >= ends[None, :]).astype(jnp.int32), axis=1),
        E - 1).astype(jnp.int32)
    be_last = jnp.max(jnp.where(valid_blk, be_raw, 0)).astype(jnp.int32)
    block_expert = jnp.where(valid_blk, be_raw, be_last)
    first_flag = jnp.concatenate([
        jnp.ones((1,), jnp.int32),
        (block_expert[1:] != block_expert[:-1]).astype(jnp.int32)])
    jb = jnp.cumsum(first_flag) - 1
    nreg = (jb[-1] + 1).reshape(1)
    seq = jnp.zeros((E,), jnp.int32).at[jb].max(block_expert)
    nvalid = (ends[-1] // BLK).astype(jnp.int32).reshape(1)
    wrep = jnp.broadcast_to(w.reshape(-1, 1), (T * K, WREP_W))

    xb = x.astype(jnp.bfloat16).astype(jnp.float32)
    ra = lax.bitcast_convert_type(xb[:, :D2], jnp.int32)
    rb = lax.bitcast_convert_type(xb[:, D2:], jnp.int32)
    xv = lax.bitcast_convert_type(
        lax.shift_right_logical(ra, 16)
        | lax.bitwise_and(rb, jnp.int32(-65536)), jnp.float32)
    xg, wg = _sc_dispatch()(xv, wrep, pos_e, pos_o, pos)
    yg = _grouped_mlp(block_expert, nvalid, jb, seq, nreg, xg,
                      gate_up_proj, down_proj, wg)
    y0, y1 = _sc_combine()(yg, pos_e, pos_o)
    out = _combine_add(y0, y1)
    return out.reshape(orig_shape)


# speedup vs baseline: 1.1073x; 1.0011x over previous
import functools

import jax
import jax.numpy as jnp
from jax import lax
from jax.experimental import pallas as pl
from jax.experimental.pallas import tpu as pltpu
from jax.experimental.pallas import tpu_sc as plsc

E = 8
D = 1024
DFF = 1024
T = 2048
K = 2

BLK = 256
NSLOT = 2
P = T * K + E * BLK
NBLOCKS = P // BLK

NC = 2
NS = 16
NW = NC * NS
TPW = T // NW
PPW = (T * K) // NW
WREP_W = 128
D2 = D // 2


def _worker_id():
    return lax.axis_index("s") * NC + lax.axis_index("c")


@functools.lru_cache(maxsize=None)
def _sc_mesh():
    return plsc.VectorSubcoreMesh(core_axis_name="c", subcore_axis_name="s",
                                  num_cores=NC, num_subcores=NS)


@functools.lru_cache(maxsize=None)
def _sc_dispatch():
    @functools.partial(
        pl.kernel,
        out_type=(
            jax.ShapeDtypeStruct((P, D2), jnp.float32),
            jax.ShapeDtypeStruct((P, WREP_W), jnp.float32),
        ),
        mesh=_sc_mesh(),
        scratch_types=(
            pltpu.VMEM((TPW, D2), jnp.float32),
            pltpu.VMEM((PPW, WREP_W), jnp.float32),
            pltpu.VMEM((TPW,), jnp.int32),
            pltpu.VMEM((TPW,), jnp.int32),
            pltpu.VMEM((PPW,), jnp.int32),
            pltpu.SemaphoreType.DMA,
        ),
    )
    def body(hidden_hbm, wrep_hbm, pos_e_hbm, pos_o_hbm, pos_hbm,
             xg_hbm, wg_hbm, tbuf, wbuf, idx_e, idx_o, idx_p, sem):
        wid = _worker_id()
        tb = wid * TPW
        pb = wid * PPW
        pltpu.sync_copy(hidden_hbm.at[pl.ds(tb, TPW)], tbuf)
        pltpu.sync_copy(pos_e_hbm.at[pl.ds(tb, TPW)], idx_e)
        pltpu.sync_copy(pos_o_hbm.at[pl.ds(tb, TPW)], idx_o)
        pltpu.async_copy(tbuf, xg_hbm.at[idx_e], sem).wait()
        pltpu.async_copy(tbuf, xg_hbm.at[idx_o], sem).wait()
        pltpu.sync_copy(wrep_hbm.at[pl.ds(pb, PPW)], wbuf)
        pltpu.sync_copy(pos_hbm.at[pl.ds(pb, PPW)], idx_p)
        pltpu.async_copy(wbuf, wg_hbm.at[idx_p], sem).wait()

    return body


@functools.lru_cache(maxsize=None)
def _sc_combine():
    @functools.partial(
        pl.kernel,
        out_type=(
            jax.ShapeDtypeStruct((T, D2), jnp.float32),
            jax.ShapeDtypeStruct((T, D2), jnp.float32),
        ),
        mesh=_sc_mesh(),
        scratch_types=(
            pltpu.VMEM((TPW, D2), jnp.float32),
            pltpu.VMEM((TPW,), jnp.int32),
            pltpu.SemaphoreType.DMA,
        ),
    )
    def body(yg_hbm, pos_e_hbm, pos_o_hbm, y0_hbm, y1_hbm, buf, idx, sem):
        wid = _worker_id()
        tb = wid * TPW
        pltpu.sync_copy(pos_e_hbm.at[pl.ds(tb, TPW)], idx)
        pltpu.async_copy(yg_hbm.at[idx], buf, sem).wait()
        pltpu.sync_copy(buf, y0_hbm.at[pl.ds(tb, TPW)])
        pltpu.sync_copy(pos_o_hbm.at[pl.ds(tb, TPW)], idx)
        pltpu.async_copy(yg_hbm.at[idx], buf, sem).wait()
        pltpu.sync_copy(buf, y1_hbm.at[pl.ds(tb, TPW)])

    return body


def _mlp_body(be_ref, nv_ref, jb_ref, seq_ref, nreg_ref,
              x_ref, gup_any, dwn_any, wp_ref, y_ref,
              gup_sl, dwn_sl, gsem, gsem2, dsem):
    i = pl.program_id(0)
    valid = i < nv_ref[0]
    j = jb_ref[i]
    first = jnp.logical_or(i == 0, be_ref[i] != be_ref[jnp.maximum(i - 1, 0)])

    def _copies(jj):
        sl = lax.rem(jj, NSLOT)
        e = seq_ref[jj]
        return (
            pltpu.make_async_copy(gup_any.at[e, pl.ds(0, DFF)],
                                  gup_sl.at[sl, pl.ds(0, DFF)], gsem.at[sl]),
            pltpu.make_async_copy(gup_any.at[e, pl.ds(DFF, DFF)],
                                  gup_sl.at[sl, pl.ds(DFF, DFF)], gsem2.at[sl]),
            pltpu.make_async_copy(dwn_any.at[e], dwn_sl.at[sl], dsem.at[sl]),
        )

    def _start(jj):
        for c in _copies(jj):
            c.start()

    def _wait(jj):
        for c in _copies(jj):
            c.wait()

    @pl.when(i == 0)
    def _():
        _start(0)
        for jj in range(1, NSLOT):
            @pl.when(nreg_ref[0] > jj)
            def _(jj=jj):
                _start(jj)

    @pl.when(jnp.logical_and(valid, first))
    def _():
        _wait(j)

        @pl.when(jnp.logical_and(j >= 1, j + NSLOT - 1 < nreg_ref[0]))
        def _():
            _start(j + NSLOT - 1)

    @pl.when(valid)
    def _():
        sl = lax.rem(j, NSLOT)
        xi = lax.bitcast_convert_type(x_ref[...], jnp.int32)
        xa = lax.bitcast_convert_type(lax.shift_left(xi, 16), jnp.float32)
        xb = lax.bitcast_convert_type(
            lax.bitwise_and(xi, jnp.int32(-65536)), jnp.float32)
        x = jnp.concatenate([xa, xb], axis=1)
        h = lax.dot_general(x, gup_sl[sl], (((1,), (1,)), ((), ())),
                            preferred_element_type=jnp.float32,
                            precision=lax.Precision.DEFAULT)
        gate = h[:, :DFF]
        up = h[:, DFF:]
        act = gate * lax.logistic(gate) * up
        y = lax.dot_general(act, dwn_sl[sl], (((1,), (1,)), ((), ())),
                            preferred_element_type=jnp.float32,
                            precision=lax.Precision.DEFAULT)
        yw = y * wp_ref[:, 0:1]
        ra = lax.bitcast_convert_type(
            yw[:, :D2].astype(jnp.bfloat16).astype(jnp.float32), jnp.int32)
        rb = lax.bitcast_convert_type(
            yw[:, D2:].astype(jnp.bfloat16).astype(jnp.float32), jnp.int32)
        y_ref[...] = lax.bitcast_convert_type(
            lax.shift_right_logical(ra, 16)
            | lax.bitwise_and(rb, jnp.int32(-65536)), jnp.float32)


def _grouped_mlp(block_expert, nvalid, jb, seq, nreg, xg, gup, dwn, wg):
    grid_spec = pltpu.PrefetchScalarGridSpec(
        num_scalar_prefetch=5,
        grid=(NBLOCKS,),
        in_specs=[
            pl.BlockSpec((BLK, D2),
                         lambda i, be, nv, *_: (jnp.minimum(i, nv[0] - 1), 0)),
            pl.BlockSpec(memory_space=pl.ANY),
            pl.BlockSpec(memory_space=pl.ANY),
            pl.BlockSpec((BLK, WREP_W),
                         lambda i, be, nv, *_: (jnp.minimum(i, nv[0] - 1), 0)),
        ],
        out_specs=pl.BlockSpec(
            (BLK, D2), lambda i, be, nv, *_: (jnp.minimum(i, nv[0] - 1), 0)),
        scratch_shapes=[
            pltpu.VMEM((NSLOT, 2 * DFF, D), jnp.float32),
            pltpu.VMEM((NSLOT, D, DFF), jnp.float32),
            pltpu.SemaphoreType.DMA((NSLOT,)),
            pltpu.SemaphoreType.DMA((NSLOT,)),
            pltpu.SemaphoreType.DMA((NSLOT,)),
        ],
    )
    return pl.pallas_call(
        _mlp_body,
        grid_spec=grid_spec,
        out_shape=jax.ShapeDtypeStruct((P, D2), jnp.float32),
    )(block_expert, nvalid, jb, seq, nreg, xg, gup, dwn, wg)


def _add_body(a_ref, b_ref, o_ref):
    q0 = lax.bitcast_convert_type(a_ref[...], jnp.int32)
    q1 = lax.bitcast_convert_type(b_ref[...], jnp.int32)

    def lo(q):
        return lax.bitcast_convert_type(lax.shift_left(q, 16), jnp.float32)

    def hi(q):
        return lax.bitcast_convert_type(
            lax.bitwise_and(q, jnp.int32(-65536)), jnp.float32)

    o_ref[:, :D2] = lo(q0) + lo(q1)
    o_ref[:, D2:] = hi(q0) + hi(q1)


def _combine_add(y0, y1):
    return pl.pallas_call(
        _add_body,
        grid=(T // 256,),
        in_specs=[
            pl.BlockSpec((256, D2), lambda i: (i, 0)),
            pl.BlockSpec((256, D2), lambda i: (i, 0)),
        ],
        out_specs=pl.BlockSpec((256, D), lambda i: (i, 0)),
        out_shape=jax.ShapeDtypeStruct((T, D), jnp.float32),
    )(y0, y1)


def kernel(hidden_states, top_k_index, top_k_weights, gate_up_proj, down_proj):
    orig_shape = hidden_states.shape
    x = hidden_states.reshape(-1, D)
    idx = top_k_index.reshape(-1, K).astype(jnp.int32)
    w = top_k_weights.reshape(-1, K).astype(jnp.float32)

    eflat = idx.reshape(-1)
    ohi = (eflat[:, None] == jnp.arange(E, dtype=jnp.int32)[None, :]).astype(jnp.int32)
    rank = jnp.sum((jnp.cumsum(ohi, axis=0) - ohi) * ohi, axis=1)
    counts = jnp.sum(ohi, axis=0)
    padded = ((counts + BLK - 1) // BLK) * BLK
    ends = jnp.cumsum(padded)
    starts = ends - padded
    pos = jnp.sum(ohi * starts[None, :], axis=1) + rank
    pos2 = pos.reshape(T, K)
    pos_e = pos2[:, 0]
    pos_o = pos2[:, 1]
    blk_start = jnp.arange(NBLOCKS, dtype=jnp.int32) * BLK
    valid_blk = blk_start < ends[-1]
    be_raw = jnp.minimum(
        jnp.sum((blk_start[:, None] >= ends[None, :]).astype(jnp.int32), axis=1),
        E - 1).astype(jnp.int32)
    be_last = jnp.max(jnp.where(valid_blk, be_raw, 0)).astype(jnp.int32)
    block_expert = jnp.where(valid_blk, be_raw, be_last)
    first_flag = jnp.concatenate([
        jnp.ones((1,), jnp.int32),
        (block_expert[1:] != block_expert[:-1]).astype(jnp.int32)])
    jb = jnp.cumsum(first_flag) - 1
    nreg = (jb[-1] + 1).reshape(1)
    seq = jnp.zeros((E,), jnp.int32).at[jb].max(block_expert)
    nvalid = (ends[-1] // BLK).astype(jnp.int32).reshape(1)
    wrep = jnp.broadcast_to(w.reshape(-1, 1), (T * K, WREP_W))

    xi_full = lax.bitcast_convert_type(x, jnp.int32)
    ra = xi_full[:, :D2]
    rb = xi_full[:, D2:]
    xv = lax.bitcast_convert_type(
        lax.shift_right_logical(ra, 16)
        | lax.bitwise_and(rb, jnp.int32(-65536)), jnp.float32)
    xg, wg = _sc_dispatch()(xv, wrep, pos_e, pos_o, pos)
    yg = _grouped_mlp(block_expert, nvalid, jb, seq, nreg, xg,
                      gate_up_proj, down_proj, wg)
    y0, y1 = _sc_combine()(yg, pos_e, pos_o)
    out = _combine_add(y0, y1)
    return out.reshape(orig_shape)
